# baseline (speedup 1.0000x reference)
import jax
import jax.numpy as jnp
from jax.experimental import pallas as pl
from jax.experimental.pallas import tpu as pltpu


BSTEP = 16


def _make_kernel(num, b, h, w, size):
    n_pool = b // BSTEP

    def body(w_ref, g_ref, m0_ref, m1_ref, out_ref, gt_ref):
        i = pl.program_id(0)

        @pl.when(i < n_pool)
        def _pool():
            s2d = (jax.lax.broadcasted_iota(jnp.int32, (w * size, 2 * w), 0)
                   // size
                   == jax.lax.broadcasted_iota(jnp.int32, (w * size, 2 * w),
                                               1) % w
                   ).astype(jnp.float32)
            cp = jnp.dot(g_ref[...], s2d, preferred_element_type=jnp.float32)
            nr = BSTEP * h * w // 128
            t = cp.reshape(nr, 2, size, 2 * w).sum(axis=2)
            lane = jax.lax.broadcasted_iota(jnp.int32, (nr, 2 * w), 1)
            gt_ref[pl.ds(i * nr, nr), :] = jnp.where(lane < w, t[:, 0, :],
                                                     t[:, 1, :])

        @pl.when(i == n_pool)
        def _loss():
            gt = gt_ref[...]
            m0 = m0_ref[...]
            m1 = m1_ref[...]
            hw128 = h * w // 128
            err0 = jnp.abs(gt - m0).reshape(b, hw128, 128)
            err1 = jnp.abs(gt - m1).reshape(b, hw128, 128)
            fnum = jnp.float32(num)

            def search_step(err, lo, hi):
                mid = lo + ((hi - lo) >> 1)
                midf = jax.lax.bitcast_convert_type(mid, jnp.float32)
                cnt = jnp.sum((err >= midf).astype(jnp.float32), axis=(1, 2),
                              keepdims=True)
                ge = cnt >= fnum
                return jnp.where(ge, mid, lo), jnp.where(ge, hi, mid)

            def step(_, carry):
                lo0, hi0, lo1, hi1 = carry
                lo0, hi0 = search_step(err0, lo0, hi0)
                lo1, hi1 = search_step(err1, lo1, hi1)
                return lo0, hi0, lo1, hi1

            z = jnp.zeros((b, 1, 1), jnp.int32)
            f = jnp.full((b, 1, 1), 0x7F800000, jnp.int32)
            lo0, hi0, lo1, hi1 = jax.lax.fori_loop(0, 31, step, (z, f, z, f))
            vmin0 = jax.lax.bitcast_convert_type(lo0, jnp.float32)
            vmin1 = jax.lax.bitcast_convert_type(lo1, jnp.float32)

            wgt = w_ref[0, 0]
            gt3 = gt.reshape(b, hw128, 128)
            m03 = m0.reshape(b, hw128, 128)
            m13 = m1.reshape(b, hw128, 128)
            comb0 = wgt * m03 + (1.0 - wgt) * gt3
            comb1 = wgt * m13 + (1.0 - wgt) * gt3
            d0 = m03 - jnp.where(err0 >= vmin0, comb1, gt3)
            d1 = m13 - jnp.where(err1 >= vmin1, comb0, gt3)
            out_ref[0, 0] = jnp.sum(d0 * d0) + jnp.sum(d1 * d1)

    return body


def kernel(dmap_conv, dmap_tran, gt_density, process):
    b, c, h, w = dmap_conv.shape
    gb, gc, gh, gw = gt_density.shape
    size = gh // h
    max_noisy_ratio = 0.1
    max_weight_ratio = 1.0
    num = int(h * w * max_noisy_ratio * 1.0)
    weight = (jnp.asarray(process, jnp.float32) * max_weight_ratio
              ).reshape(1, 1)

    m0 = dmap_conv.reshape(b * h * w // 128, 128)
    m1 = dmap_tran.reshape(b * h * w // 128, 128)
    g2 = gt_density.reshape(gb * gh, gw)

    out = pl.pallas_call(
        _make_kernel(num, b, h, w, size),
        grid=(b // BSTEP + 1,),
        in_specs=[
            pl.BlockSpec(memory_space=pltpu.SMEM),
            pl.BlockSpec((BSTEP * gh, gw),
                         lambda i: (jnp.minimum(i, b // BSTEP - 1), 0)),
            pl.BlockSpec((b * h * w // 128, 128), lambda i: (0, 0)),
            pl.BlockSpec((b * h * w // 128, 128), lambda i: (0, 0)),
        ],
        out_specs=pl.BlockSpec(memory_space=pltpu.SMEM),
        out_shape=jax.ShapeDtypeStruct((1, 1), jnp.float32),
        scratch_shapes=[pltpu.VMEM((b * h * w // 128, 128), jnp.float32)],
    )(weight, g2, m0, m1)
    return out.reshape(())


# speedup vs baseline: 1.0476x; 1.0476x over previous
import jax
import jax.numpy as jnp
from jax.experimental import pallas as pl
from jax.experimental.pallas import tpu as pltpu


BSTEP = 8


def _make_kernel(num, b, h, w, size):
    n_pool = b // BSTEP

    def body(w_ref, g_ref, m0_ref, m1_ref, out_ref, gt_ref):
        i = pl.program_id(0)

        @pl.when(i < n_pool)
        def _pool():
            s2d = (jax.lax.broadcasted_iota(jnp.int32, (w * size, 2 * w), 0)
                   // size
                   == jax.lax.broadcasted_iota(jnp.int32, (w * size, 2 * w),
                                               1) % w
                   ).astype(jnp.float32)
            cp = jnp.dot(g_ref[...], s2d, preferred_element_type=jnp.float32)
            nr = BSTEP * h * w // 128
            t = cp.reshape(nr, 2, size, 2 * w).sum(axis=2)
            lane = jax.lax.broadcasted_iota(jnp.int32, (nr, 2 * w), 1)
            gt_ref[pl.ds(i * nr, nr), :] = jnp.where(lane < w, t[:, 0, :],
                                                     t[:, 1, :])

        @pl.when(i == n_pool)
        def _loss():
            gt = gt_ref[...]
            m0 = m0_ref[...]
            m1 = m1_ref[...]
            hw128 = h * w // 128
            err0 = jnp.abs(gt - m0).reshape(b, hw128, 128)
            err1 = jnp.abs(gt - m1).reshape(b, hw128, 128)
            fnum = jnp.float32(num)

            def search_step(err, lo, hi):
                mid = lo + ((hi - lo) >> 1)
                midf = jax.lax.bitcast_convert_type(mid, jnp.float32)
                cnt = jnp.sum((err >= midf).astype(jnp.float32), axis=(1, 2),
                              keepdims=True)
                ge = cnt >= fnum
                return jnp.where(ge, mid, lo), jnp.where(ge, hi, mid)

            def step(_, carry):
                lo0, hi0, lo1, hi1 = carry
                lo0, hi0 = search_step(err0, lo0, hi0)
                lo1, hi1 = search_step(err1, lo1, hi1)
                return lo0, hi0, lo1, hi1

            z = jnp.zeros((b, 1, 1), jnp.int32)
            f = jnp.full((b, 1, 1), 0x7F800000, jnp.int32)
            lo0, hi0, lo1, hi1 = jax.lax.fori_loop(0, 31, step, (z, f, z, f))
            vmin0 = jax.lax.bitcast_convert_type(lo0, jnp.float32)
            vmin1 = jax.lax.bitcast_convert_type(lo1, jnp.float32)

            wgt = w_ref[0, 0]
            gt3 = gt.reshape(b, hw128, 128)
            m03 = m0.reshape(b, hw128, 128)
            m13 = m1.reshape(b, hw128, 128)
            comb0 = wgt * m03 + (1.0 - wgt) * gt3
            comb1 = wgt * m13 + (1.0 - wgt) * gt3
            d0 = m03 - jnp.where(err0 >= vmin0, comb1, gt3)
            d1 = m13 - jnp.where(err1 >= vmin1, comb0, gt3)
            out_ref[0, 0] = jnp.sum(d0 * d0) + jnp.sum(d1 * d1)

    return body


def kernel(dmap_conv, dmap_tran, gt_density, process):
    b, c, h, w = dmap_conv.shape
    gb, gc, gh, gw = gt_density.shape
    size = gh // h
    max_noisy_ratio = 0.1
    max_weight_ratio = 1.0
    num = int(h * w * max_noisy_ratio * 1.0)
    weight = (jnp.asarray(process, jnp.float32) * max_weight_ratio
              ).reshape(1, 1)

    m0 = dmap_conv.reshape(b * h * w // 128, 128)
    m1 = dmap_tran.reshape(b * h * w // 128, 128)
    g2 = gt_density.reshape(gb * gh, gw)

    out = pl.pallas_call(
        _make_kernel(num, b, h, w, size),
        grid=(b // BSTEP + 1,),
        in_specs=[
            pl.BlockSpec(memory_space=pltpu.SMEM),
            pl.BlockSpec((BSTEP * gh, gw),
                         lambda i: (jnp.minimum(i, b // BSTEP - 1), 0)),
            pl.BlockSpec((b * h * w // 128, 128), lambda i: (0, 0)),
            pl.BlockSpec((b * h * w // 128, 128), lambda i: (0, 0)),
        ],
        out_specs=pl.BlockSpec(memory_space=pltpu.SMEM),
        out_shape=jax.ShapeDtypeStruct((1, 1), jnp.float32),
        scratch_shapes=[pltpu.VMEM((b * h * w // 128, 128), jnp.float32)],
    )(weight, g2, m0, m1)
    return out.reshape(())
